# baseline (speedup 1.0000x reference)
import functools

import jax
import jax.numpy as jnp
from jax import lax
from jax.experimental import pallas as pl
from jax.experimental.pallas import tpu as pltpu
from jax.experimental.pallas import tpu_sc as plsc

NC = 2
NS = 16
L = 16
NW = NC * NS

B = 16384 * 26
D = 64
PER_W = B // NW
NIDX = 128
CHUNK = 512
GPC = CHUNK // NIDX
NCHUNK = PER_W // CHUNK
IDX_ROWS = PER_W // NIDX


def _body(x_hbm, idx_hbm, tab_hbm, out_hbm, idx_v, rows_v, x_v, gsem, xsem):
    wid = lax.axis_index("s") * NC + lax.axis_index("c")
    base = wid * PER_W

    pltpu.sync_copy(idx_hbm.at[pl.ds(wid * IDX_ROWS, IDX_ROWS)], idx_v)

    def chunk_body(c, carry):
        rbase = base + c * CHUNK
        xcp = pltpu.async_copy(x_hbm.at[pl.ds(rbase, CHUNK)], x_v, xsem)
        gcps = []
        for j in range(GPC):
            gcps.append(
                pltpu.async_copy(
                    tab_hbm.at[idx_v.at[c * GPC + j]],
                    rows_v.at[pl.ds(j * NIDX, NIDX)],
                    gsem,
                )
            )
        xcp.wait()
        for g in gcps:
            g.wait()

        def row_body(r, carry2):
            for d in range(D // L):
                s = pl.ds(d * L, L)
                rows_v[r, s] = rows_v[r, s] + x_v[r, s]
            return carry2

        lax.fori_loop(0, CHUNK, row_body, 0, unroll=2)
        pltpu.sync_copy(rows_v, out_hbm.at[pl.ds(rbase, CHUNK)])
        return carry

    lax.fori_loop(0, NCHUNK, chunk_body, 0)


_sc_call = functools.partial(
    pl.kernel,
    mesh=plsc.VectorSubcoreMesh(core_axis_name="c", subcore_axis_name="s"),
    out_type=jax.ShapeDtypeStruct((B, D), jnp.float32),
    scratch_types=[
        pltpu.VMEM((IDX_ROWS, NIDX), jnp.int32),
        pltpu.VMEM((CHUNK, D), jnp.float32),
        pltpu.VMEM((CHUNK, D), jnp.float32),
        pltpu.SemaphoreType.DMA,
        pltpu.SemaphoreType.DMA,
    ],
    compiler_params=pltpu.CompilerParams(use_tc_tiling_on_sc=False),
)(_body)


@jax.jit
def kernel(x, idxs, embed_weight):
    xf = x.reshape(B, D)
    idxf = idxs.astype(jnp.int32).reshape(B // NIDX, NIDX)
    out = _sc_call(xf, idxf, embed_weight)
    return out.reshape(x.shape)


# speedup vs baseline: 1.1876x; 1.1876x over previous
import functools

import jax
import jax.numpy as jnp
from jax import lax
from jax.experimental import pallas as pl
from jax.experimental.pallas import tpu as pltpu
from jax.experimental.pallas import tpu_sc as plsc

NC = 2
NS = 16
L = 16
NW = NC * NS

B = 16384 * 26
D = 64
PER_W = B // NW
NIDX = 128
CHUNK = 512
GPC = CHUNK // NIDX
NCHUNK = PER_W // CHUNK
IDX_ROWS = PER_W // NIDX


def _body(x_hbm, idx_hbm, tab_hbm, out_hbm, idx_v, rows_v, x_v, gsem, xsem):
    wid = lax.axis_index("s") * NC + lax.axis_index("c")
    base = wid * PER_W

    pltpu.sync_copy(idx_hbm.at[pl.ds(wid * IDX_ROWS, IDX_ROWS)], idx_v)

    def chunk_body(c, carry):
        rbase = base + c * CHUNK
        pltpu.sync_copy(x_hbm.at[pl.ds(rbase, CHUNK)], rows_v)
        gcps = []
        for j in range(GPC):
            gcps.append(
                pltpu.async_copy(
                    tab_hbm.at[idx_v.at[c * GPC + j]],
                    rows_v.at[pl.ds(j * NIDX, NIDX)],
                    gsem,
                    add=True,
                )
            )
        for g in gcps:
            g.wait()
        pltpu.sync_copy(rows_v, out_hbm.at[pl.ds(rbase, CHUNK)])
        return carry

    lax.fori_loop(0, NCHUNK, chunk_body, 0)


_sc_call = functools.partial(
    pl.kernel,
    mesh=plsc.VectorSubcoreMesh(core_axis_name="c", subcore_axis_name="s"),
    out_type=jax.ShapeDtypeStruct((B, D), jnp.float32),
    scratch_types=[
        pltpu.VMEM((IDX_ROWS, NIDX), jnp.int32),
        pltpu.VMEM((CHUNK, D), jnp.float32),
        pltpu.VMEM((CHUNK, D), jnp.float32),
        pltpu.SemaphoreType.DMA,
        pltpu.SemaphoreType.DMA,
    ],
    compiler_params=pltpu.CompilerParams(use_tc_tiling_on_sc=False),
)(_body)


@jax.jit
def kernel(x, idxs, embed_weight):
    xf = x.reshape(B, D)
    idxf = idxs.astype(jnp.int32).reshape(B // NIDX, NIDX)
    out = _sc_call(xf, idxf, embed_weight)
    return out.reshape(x.shape)


# speedup vs baseline: 1.2151x; 1.0231x over previous
import functools

import jax
import jax.numpy as jnp
from jax import lax
from jax.experimental import pallas as pl
from jax.experimental.pallas import tpu as pltpu
from jax.experimental.pallas import tpu_sc as plsc

NC = 2
NS = 16
NW = NC * NS

B = 16384 * 26
D = 64
PER_W = B // NW
NIDX = 128
CHUNK = 512
GPC = CHUNK // NIDX
NCHUNK = PER_W // CHUNK
IDX_ROWS = PER_W // NIDX
NBUF = 3


def _body(x_hbm, idx_hbm, tab_hbm, out_hbm, idx_v, buf0, buf1, buf2,
          xsem, gsem, osem):
    wid = lax.axis_index("s") * NC + lax.axis_index("c")
    base = wid * PER_W
    bufs = (buf0, buf1, buf2)

    pltpu.sync_copy(idx_hbm.at[pl.ds(wid * IDX_ROWS, IDX_ROWS)], idx_v)

    def issue_x(c):
        pltpu.async_copy(
            x_hbm.at[pl.ds(base + c * CHUNK, CHUNK)], bufs[c % NBUF], xsem)

    def wait_x():
        pltpu.make_async_copy(
            x_hbm.at[pl.ds(base, CHUNK)], buf0, xsem).wait()

    def wait_out():
        pltpu.make_async_copy(
            x_hbm.at[pl.ds(base, CHUNK)], buf0, osem).wait()

    issue_x(0)
    issue_x(1)

    for c in range(NCHUNK):
        b = bufs[c % NBUF]
        wait_x()
        gcps = []
        for j in range(GPC):
            gcps.append(
                pltpu.async_copy(
                    tab_hbm.at[idx_v.at[c * GPC + j]],
                    b.at[pl.ds(j * NIDX, NIDX)],
                    gsem,
                    add=True,
                )
            )
        for g in gcps:
            g.wait()
        pltpu.async_copy(b, out_hbm.at[pl.ds(base + c * CHUNK, CHUNK)], osem)
        if c >= 1:
            wait_out()
        if c + 2 < NCHUNK:
            issue_x(c + 2)
    wait_out()


_sc_call = functools.partial(
    pl.kernel,
    mesh=plsc.VectorSubcoreMesh(core_axis_name="c", subcore_axis_name="s"),
    out_type=jax.ShapeDtypeStruct((B, D), jnp.float32),
    scratch_types=[
        pltpu.VMEM((IDX_ROWS, NIDX), jnp.int32),
        pltpu.VMEM((CHUNK, D), jnp.float32),
        pltpu.VMEM((CHUNK, D), jnp.float32),
        pltpu.VMEM((CHUNK, D), jnp.float32),
        pltpu.SemaphoreType.DMA,
        pltpu.SemaphoreType.DMA,
        pltpu.SemaphoreType.DMA,
    ],
    compiler_params=pltpu.CompilerParams(use_tc_tiling_on_sc=False),
)(_body)


@jax.jit
def kernel(x, idxs, embed_weight):
    xf = x.reshape(B, D)
    idxf = idxs.astype(jnp.int32).reshape(B // NIDX, NIDX)
    out = _sc_call(xf, idxf, embed_weight)
    return out.reshape(x.shape)
